# baseline (speedup 1.0000x reference)
import functools

import jax
import jax.numpy as jnp
from jax import lax
from jax.experimental import pallas as pl
from jax.experimental.pallas import tpu as pltpu
from jax.experimental.pallas import tpu_sc as plsc

_INFO = plsc.get_sparse_core_info()
_NC, _NS, _L = _INFO.num_cores, _INFO.num_subcores, _INFO.num_lanes
_NW = _NC * _NS

_IDX_W = 128
_CHUNK = 1280
_G = _CHUNK // _IDX_W


def _make_kernel(N, V, D):
    assert N % (_NW * _CHUNK) == 0
    per_w = N // _NW
    n_chunks = per_w // _CHUNK

    @functools.partial(
        pl.kernel,
        out_type=jax.ShapeDtypeStruct((N, D), jnp.float32),
        mesh=plsc.VectorSubcoreMesh(core_axis_name="c", subcore_axis_name="s"),
        scratch_types=[
            pltpu.VMEM((_CHUNK,), jnp.int32),
            pltpu.VMEM((_CHUNK,), jnp.float32),
            pltpu.VMEM((_CHUNK, D), jnp.float32),
            pltpu.SemaphoreType.DMA,
        ],
        compiler_params=pltpu.CompilerParams(use_tc_tiling_on_sc=False),
    )
    def k(table_hbm, x_hbm, xval_hbm, out_hbm, idx_v, xval_v, rows_v, sem):
        wid = lax.axis_index("s") * _NC + lax.axis_index("c")
        base = wid * per_w

        def chunk_body(c, carry):
            cbase = base + c * _CHUNK
            pltpu.sync_copy(x_hbm.at[pl.ds(cbase, _CHUNK)], idx_v)
            pltpu.sync_copy(xval_hbm.at[pl.ds(cbase, _CHUNK)], xval_v)
            copies = [
                pltpu.async_copy(
                    table_hbm.at[idx_v.at[pl.ds(j * _IDX_W, _IDX_W)]],
                    rows_v.at[pl.ds(j * _IDX_W, _IDX_W)],
                    sem,
                )
                for j in range(_G)
            ]
            for cp in copies:
                cp.wait()

            def grp_body(g, _):
                sv = xval_v[pl.ds(g * _L, _L)]
                for kk in range(_L):
                    r = g * _L + kk
                    s = jnp.full((_L,), sv[kk], jnp.float32)
                    for h in range(D // _L):
                        rows_v[r, pl.ds(h * _L, _L)] = (
                            rows_v[r, pl.ds(h * _L, _L)] * s
                        )
                return 0

            lax.fori_loop(0, _CHUNK // _L, grp_body, 0)
            pltpu.sync_copy(rows_v, out_hbm.at[pl.ds(cbase, _CHUNK)])
            return carry

        lax.fori_loop(0, n_chunks, chunk_body, 0)

    return k


@jax.jit
def kernel(x, x_val, table):
    B, NNZ = x.shape
    V, D = table.shape
    N = B * NNZ
    xf = x.reshape(N).astype(jnp.int32)
    vf = x_val.reshape(N)
    out = _make_kernel(N, V, D)(table, xf, vf)
    return out.reshape(B, NNZ, D)


# speedup vs baseline: 1.2488x; 1.2488x over previous
import functools

import jax
import jax.numpy as jnp
from jax import lax
from jax.experimental import pallas as pl
from jax.experimental.pallas import tpu as pltpu
from jax.experimental.pallas import tpu_sc as plsc

_INFO = plsc.get_sparse_core_info()
_NC, _NS, _L = _INFO.num_cores, _INFO.num_subcores, _INFO.num_lanes
_NW = _NC * _NS

_G = 16


def _make_kernel(B, NNZ, V, D):
    assert B % _NW == 0
    rows_per_w = B // _NW
    assert rows_per_w % _G == 0
    n_chunks = rows_per_w // _G
    n_full = NNZ // _L
    n_tail = NNZ - n_full * _L

    @functools.partial(
        pl.kernel,
        out_type=jax.ShapeDtypeStruct((B, NNZ, D), jnp.float32),
        mesh=plsc.VectorSubcoreMesh(core_axis_name="c", subcore_axis_name="s"),
        scratch_types=[
            pltpu.VMEM((_G, NNZ), jnp.int32),
            pltpu.VMEM((_G, NNZ), jnp.float32),
            pltpu.VMEM((_G, NNZ, D), jnp.float32),
            pltpu.SemaphoreType.DMA,
        ],
        compiler_params=pltpu.CompilerParams(
            use_tc_tiling_on_sc=False, needs_layout_passes=False
        ),
    )
    def k(table_hbm, x_hbm, xval_hbm, out_hbm, idx_v, xval_v, rows_v, sem):
        wid = lax.axis_index("s") * _NC + lax.axis_index("c")
        base = wid * rows_per_w

        def scale_rows(g):
            def do_row(n, s):
                for c in range(D // _L):
                    rows_v[g, n, pl.ds(c * _L, _L)] = (
                        rows_v[g, n, pl.ds(c * _L, _L)] * s
                    )

            for h in range(n_full):
                sv = xval_v[g, pl.ds(h * _L, _L)]
                for kk in range(_L):
                    do_row(h * _L + kk, jnp.full((_L,), sv[kk], jnp.float32))
            if n_tail:
                nv = jnp.minimum(
                    lax.iota(jnp.int32, _L) + (NNZ - n_tail), NNZ - 1
                )
                gv = jnp.full((_L,), g, jnp.int32)
                sv = plsc.load_gather(xval_v, [gv, nv])
                for kk in range(n_tail):
                    do_row(
                        NNZ - n_tail + kk,
                        jnp.full((_L,), sv[kk], jnp.float32),
                    )

        def chunk_body(ci, carry):
            b0 = base + ci * _G
            pltpu.sync_copy(x_hbm.at[pl.ds(b0, _G)], idx_v)
            pltpu.sync_copy(xval_hbm.at[pl.ds(b0, _G)], xval_v)
            copies = [
                pltpu.async_copy(
                    table_hbm.at[idx_v.at[g]], rows_v.at[g], sem
                )
                for g in range(_G)
            ]
            for cp in copies:
                cp.wait()

            def g_body(g, c2):
                scale_rows(g)
                return c2

            lax.fori_loop(0, _G, g_body, 0)
            pltpu.sync_copy(rows_v, out_hbm.at[pl.ds(b0, _G)])
            return carry

        lax.fori_loop(0, n_chunks, chunk_body, 0)

    return k


@jax.jit
def kernel(x, x_val, table):
    B, NNZ = x.shape
    V, D = table.shape
    return _make_kernel(B, NNZ, V, D)(table, x.astype(jnp.int32), x_val)
